# baseline (speedup 1.0000x reference)
import jax
import jax.numpy as jnp
from jax import lax
from jax.experimental import pallas as pl
from jax.experimental.pallas import tpu as pltpu
from jax.experimental.pallas import tpu_sc as plsc

N_NODES = 10000
N_EDGES = 320000
ND = 128
ED = 16

NC, NS = 2, 16
NW = NC * NS
CHE = 128
NCHK = N_EDGES // CHE
CPW = NCHK // NW
NB = 4
RPT = N_NODES // NS

_EPS = 1e-5


def _pq_body(x8_ref, wa_ref, wb_ref, p_ref, q_ref):
    x8 = x8_ref[...]
    p_ref[...] = jnp.dot(x8, wa_ref[...], preferred_element_type=jnp.float32)
    q_ref[...] = jnp.dot(x8, wb_ref[...], preferred_element_type=jnp.float32)


def _project_pq(x8, w8a, w8b):
    r = N_NODES // 8
    return pl.pallas_call(
        _pq_body,
        grid=(1,),
        in_specs=[
            pl.BlockSpec((r, 8 * ND), lambda m: (0, 0)),
            pl.BlockSpec((8 * ND, ND), lambda m: (0, 0)),
            pl.BlockSpec((8 * ND, ND), lambda m: (0, 0)),
        ],
        out_specs=[
            pl.BlockSpec((r, ND), lambda m: (0, 0)),
            pl.BlockSpec((r, ND), lambda m: (0, 0)),
        ],
        out_shape=[
            jax.ShapeDtypeStruct((r, ND), jnp.float32),
            jax.ShapeDtypeStruct((r, ND), jnp.float32),
        ],
    )(x8, w8a, w8b)


def _worker_chunks():
    wid = lax.axis_index("s") * NC + lax.axis_index("c")
    cnt = CPW + jnp.where(wid < 4, 1, 0)
    cbase = wid * CPW + jnp.minimum(wid, 4)
    cstart = jnp.minimum(cbase, NCHK - (CPW + 1))
    off = cbase - cstart
    return cnt, cbase, off


def _iota16():
    return lax.iota(jnp.int32, 16)


def _gather_body(p_hbm, q_hbm, ei_hbm, out_hbm,
                 idx_v, ra, rb, patch,
                 sa0, sa1, sa2, sa3, sb0, sb1, sb2, sb3,
                 so0, so1, so2, so3):
    sas = (sa0, sa1, sa2, sa3)
    sbs = (sb0, sb1, sb2, sb3)
    sos = (so0, so1, so2, so3)
    cnt, cbase, off = _worker_chunks()
    pltpu.sync_copy(ei_hbm.at[pl.ds(cbase - off, CPW + 1)], idx_v)
    it = _iota16()

    def issue(t, b):
        @pl.when(t < cnt)
        def _():
            pltpu.async_copy(p_hbm.at[idx_v.at[off + t, 0]], ra.at[b], sas[b])
            pltpu.async_copy(q_hbm.at[idx_v.at[off + t, 1]], rb.at[b], sbs[b])

    def drain(d, b):
        @pl.when(d < cnt)
        def _():
            pltpu.make_async_copy(
                p_hbm.at[pl.ds(0, CHE)], ra.at[b], sas[b]).wait()
            pltpu.make_async_copy(
                q_hbm.at[pl.ds(0, CHE)], rb.at[b], sbs[b]).wait()

            @pl.when(d >= NB)
            def _():
                pltpu.make_async_copy(
                    patch.at[b, 0], out_hbm.at[0, 0], sos[b]).wait()
                pltpu.make_async_copy(
                    patch.at[b, 1], out_hbm.at[1, 0], sos[b]).wait()
            rav, rbv = ra.at[b], rb.at[b]
            pav = patch.at[b]
            for f in range(ED):
                fcol = jnp.full((16,), f, jnp.int32)
                for grp in range(8):
                    rows = grp * 16 + it
                    va = plsc.load_gather(rav, [rows, fcol])
                    vb = plsc.load_gather(rbv, [rows, fcol])
                    pav[f // 8, f % 8, pl.ds(grp * 16, 16)] = va + vb
            cb = cbase + d
            pltpu.async_copy(pav.at[0], out_hbm.at[0, cb], sos[b])
            pltpu.async_copy(pav.at[1], out_hbm.at[1, cb], sos[b])

    for b in range(NB - 1):
        issue(b, b)

    def body(g, c):
        for u in range(NB):
            t = g * NB + u
            issue(t + (NB - 1), (u + NB - 1) % NB)
            drain(t, u)
        return c

    lax.fori_loop(0, (CPW + NB) // NB, body, 0)
    for b in range(NB):
        pltpu.make_async_copy(
            patch.at[b, 0], out_hbm.at[0, 0], sos[b]).wait()
        pltpu.make_async_copy(
            patch.at[b, 1], out_hbm.at[1, 0], sos[b]).wait()


def _gather_pq(p, q, ei4):
    f = pl.kernel(
        _gather_body,
        out_type=jax.ShapeDtypeStruct((2, NCHK, 8, ND), jnp.float32),
        mesh=plsc.VectorSubcoreMesh(
            core_axis_name="c", subcore_axis_name="s",
            num_cores=NC, num_subcores=NS),
        scratch_types=[
            pltpu.VMEM((CPW + 1, 2, CHE), jnp.int32),
            pltpu.VMEM((NB, CHE, ED), jnp.float32),
            pltpu.VMEM((NB, CHE, ED), jnp.float32),
            pltpu.VMEM((NB, 2, 8, ND), jnp.float32),
        ] + [pltpu.SemaphoreType.DMA] * (3 * NB),
        compiler_params=pltpu.CompilerParams(use_tc_tiling_on_sc=False, needs_layout_passes=False),
    )
    return f(p, q, ei4)


_CBLK = 50


def _edge_body(g4_ref, eat_ref, we_ref, w2_ref, w3_ref, vb_ref, outt_ref):
    pieces = [g4_ref[:, cb].reshape(ED, ND) for cb in range(_CBLK)]
    g = jnp.concatenate(pieces, axis=1)
    ea = eat_ref[...]
    vb = vb_ref[...]
    b1 = vb[:, 0:1]
    b2 = vb[:, 1:2]
    b3 = vb[:, 2:3]
    gam = vb[:, 3:4]
    bet = vb[:, 4:5]
    t = g + jnp.dot(we_ref[...], ea, preferred_element_type=jnp.float32) + b1
    t = t * jax.nn.sigmoid(t)
    t = jnp.dot(w2_ref[...], t, preferred_element_type=jnp.float32) + b2
    t = t * jax.nn.sigmoid(t)
    t = jnp.dot(w3_ref[...], t, preferred_element_type=jnp.float32) + b3
    mu = jnp.mean(t, axis=0, keepdims=True)
    d = t - mu
    var = jnp.mean(d * d, axis=0, keepdims=True)
    outt_ref[...] = ea + d * lax.rsqrt(var + _EPS) * gam + bet


def _edge_mlp(g4, ea_t, wet, w2t, w3t, vb):
    ne_blk = _CBLK * ND
    full16 = pl.BlockSpec((ED, ED), lambda m: (0, 0))
    return pl.pallas_call(
        _edge_body,
        grid=(NCHK // _CBLK,),
        in_specs=[
            pl.BlockSpec((2, _CBLK, 8, ND), lambda m: (0, m, 0, 0)),
            pl.BlockSpec((ED, ne_blk), lambda m: (0, m)),
            full16, full16, full16,
            pl.BlockSpec((ED, 8), lambda m: (0, 0)),
        ],
        out_specs=pl.BlockSpec((ED, ne_blk), lambda m: (0, m)),
        out_shape=jax.ShapeDtypeStruct((ED, N_EDGES), jnp.float32),
    )(g4, ea_t, wet, w2t, w3t, vb)


def _scatter_body(e4_hbm, ei_hbm, out_hbm,
                  idx_v, ebuf, rbuf, zbuf, acc,
                  se0, se1, se2, se3, ss0, ss1, ss2, ss3):
    ses = (se0, se1, se2, se3)
    sss = (ss0, ss1, ss2, ss3)
    cid = lax.axis_index("c")
    sid = lax.axis_index("s")
    cnt, cbase, off = _worker_chunks()
    it = _iota16()

    def zb(r, c):
        zbuf[r] = jnp.zeros((ED,), jnp.float32)
        return c
    lax.fori_loop(0, 125, zb, 0, unroll=4)
    for k in range(RPT // 125):
        pltpu.sync_copy(zbuf, acc.at[pl.ds(sid * RPT + k * 125, 125)])
    plsc.subcore_barrier()

    pltpu.sync_copy(ei_hbm.at[pl.ds(cbase - off, CPW + 1)], idx_v)

    def issue(t, b):
        @pl.when(t < cnt)
        def _():
            cb = cbase + t
            pltpu.async_copy(e4_hbm.at[0, cb], ebuf.at[b, 0], ses[b])
            pltpu.async_copy(e4_hbm.at[1, cb], ebuf.at[b, 1], ses[b])

    def drain(d, b):
        @pl.when(d < cnt)
        def _():
            pltpu.make_async_copy(
                e4_hbm.at[0, 0], ebuf.at[b, 0], ses[b]).wait()
            pltpu.make_async_copy(
                e4_hbm.at[1, 0], ebuf.at[b, 1], ses[b]).wait()

            @pl.when(d >= NB)
            def _():
                pltpu.make_async_copy(
                    rbuf.at[b], acc.at[idx_v.at[0, 1]], sss[b]).wait()
            ebv, rbv = ebuf.at[b], rbuf.at[b]
            for f in range(ED):
                fdv = jnp.full((16,), f // 8, jnp.int32)
                fmv = jnp.full((16,), f % 8, jnp.int32)
                fcol = jnp.full((16,), f, jnp.int32)
                for grp in range(8):
                    lanes = grp * 16 + it
                    v = plsc.load_gather(ebv, [fdv, fmv, lanes])
                    plsc.store_scatter(rbv, [lanes, fcol], v)
            pltpu.async_copy(rbv, acc.at[idx_v.at[off + d, 1]], sss[b],
                             add=True)

    for b in range(NB - 1):
        issue(b, b)

    def body(g, c):
        for u in range(NB):
            t = g * NB + u
            issue(t + (NB - 1), (u + NB - 1) % NB)
            drain(t, u)
        return c

    lax.fori_loop(0, (CPW + NB) // NB, body, 0)
    for b in range(NB):
        pltpu.make_async_copy(
            rbuf.at[b], acc.at[idx_v.at[0, 1]], sss[b]).wait()

    plsc.subcore_barrier()
    pltpu.sync_copy(acc.at[pl.ds(sid * RPT, RPT)],
                    out_hbm.at[cid, pl.ds(sid * RPT, RPT)])


def _scatter_add(e4, ei4):
    f = pl.kernel(
        _scatter_body,
        out_type=jax.ShapeDtypeStruct((NC, N_NODES, ED), jnp.float32),
        mesh=plsc.VectorSubcoreMesh(
            core_axis_name="c", subcore_axis_name="s",
            num_cores=NC, num_subcores=NS),
        scratch_types=[
            pltpu.VMEM((CPW + 1, 2, CHE), jnp.int32),
            pltpu.VMEM((NB, 2, 8, ND), jnp.float32),
            pltpu.VMEM((NB, CHE, ED), jnp.float32),
            pltpu.VMEM((125, ED), jnp.float32),
            pltpu.VMEM_SHARED((N_NODES, ED), jnp.float32),
        ] + [pltpu.SemaphoreType.DMA] * (2 * NB),
        compiler_params=pltpu.CompilerParams(use_tc_tiling_on_sc=False, needs_layout_passes=False),
    )
    return f(e4, ei4)


def _node_body(x_ref, a0_ref, a1_ref, w1a_ref, w1b_ref, b1_ref,
               w2_ref, b2_ref, w3_ref, b3_ref, gam_ref, bet_ref, out_ref):
    x = x_ref[...]
    agg = a0_ref[...] + a1_ref[...]
    t = (jnp.dot(x, w1a_ref[...], preferred_element_type=jnp.float32)
         + jnp.dot(agg, w1b_ref[...], preferred_element_type=jnp.float32)
         + b1_ref[...])
    t = t * jax.nn.sigmoid(t)
    t = jnp.dot(t, w2_ref[...], preferred_element_type=jnp.float32) + b2_ref[...]
    t = t * jax.nn.sigmoid(t)
    t = jnp.dot(t, w3_ref[...], preferred_element_type=jnp.float32) + b3_ref[...]
    mu = jnp.mean(t, axis=-1, keepdims=True)
    d = t - mu
    var = jnp.mean(d * d, axis=-1, keepdims=True)
    out_ref[...] = x + d * lax.rsqrt(var + _EPS) * gam_ref[...] + bet_ref[...]


def _node_mlp(x, a0, a1, w1a, w1b, b1, w2, b2, w3, b3, gam, bet):
    blk = 1000
    full = pl.BlockSpec((ND, ND), lambda m: (0, 0))
    vec = pl.BlockSpec((1, ND), lambda m: (0, 0))
    return pl.pallas_call(
        _node_body,
        grid=(N_NODES // blk,),
        in_specs=[
            pl.BlockSpec((blk, ND), lambda m: (m, 0)),
            pl.BlockSpec((blk, ED), lambda m: (m, 0)),
            pl.BlockSpec((blk, ED), lambda m: (m, 0)),
            full,
            pl.BlockSpec((ED, ND), lambda m: (0, 0)),
            vec, full, vec, full, vec, vec, vec,
        ],
        out_specs=pl.BlockSpec((blk, ND), lambda m: (m, 0)),
        out_shape=jax.ShapeDtypeStruct((N_NODES, ND), jnp.float32),
    )(x, a0, a1, w1a, w1b, b1, w2, b2, w3, b3, gam, bet)


def kernel(x, edge_index, edge_attr,
           eW1, eb1, eW2, eb2, eW3, eb3, e_gamma, e_beta,
           nW1, nb1, nW2, nb2, nW3, nb3, n_gamma, n_beta):
    ei4 = edge_index.astype(jnp.int32).reshape(2, NCHK, CHE).transpose(1, 0, 2)
    eye8 = jnp.eye(8, dtype=jnp.float32)

    x8 = x.reshape(N_NODES // 8, 8 * ND)
    w8a = jnp.kron(eye8, eW1[:ND])
    w8b = jnp.kron(eye8, eW1[ND:2 * ND])
    p_pk, q_pk = _project_pq(x8, w8a, w8b)

    g4 = _gather_pq(p_pk.reshape(N_NODES, ED), q_pk.reshape(N_NODES, ED), ei4)

    vb = jnp.stack([eb1, eb2, eb3, e_gamma, e_beta], axis=1)
    vb = jnp.concatenate([vb, jnp.zeros((ED, 3), jnp.float32)], axis=1)
    e_new_t = _edge_mlp(g4, edge_attr.T, eW1[2 * ND:].T, eW2.T, eW3.T, vb)

    e4 = e_new_t.reshape(2, 8, NCHK, CHE).transpose(0, 2, 1, 3)
    aggp = _scatter_add(e4, ei4)

    x_new = _node_mlp(
        x, aggp[0], aggp[1],
        nW1[:ND], nW1[ND:], nb1.reshape(1, ND),
        nW2, nb2.reshape(1, ND), nW3, nb3.reshape(1, ND),
        n_gamma.reshape(1, ND), n_beta.reshape(1, ND))

    return (x_new, e_new_t.T)


# speedup vs baseline: 1.5351x; 1.5351x over previous
import jax
import jax.numpy as jnp
from jax import lax
from jax.experimental import pallas as pl
from jax.experimental.pallas import tpu as pltpu
from jax.experimental.pallas import tpu_sc as plsc

N_NODES = 10000
N_EDGES = 320000
ND = 128
ED = 16

NC, NS = 2, 16
NW = NC * NS
CHE = 128
NCHK = N_EDGES // CHE
CPW = NCHK // NW
NB = 4
RPT = N_NODES // NS

_EPS = 1e-5


def _pq_body(x8_ref, wa_ref, wb_ref, p_ref, q_ref):
    x8 = x8_ref[...]
    p_ref[...] = jnp.dot(x8, wa_ref[...], preferred_element_type=jnp.float32)
    q_ref[...] = jnp.dot(x8, wb_ref[...], preferred_element_type=jnp.float32)


def _project_pq(x8, w8a, w8b):
    r = N_NODES // 8
    return pl.pallas_call(
        _pq_body,
        grid=(1,),
        in_specs=[
            pl.BlockSpec((r, 8 * ND), lambda m: (0, 0)),
            pl.BlockSpec((8 * ND, ND), lambda m: (0, 0)),
            pl.BlockSpec((8 * ND, ND), lambda m: (0, 0)),
        ],
        out_specs=[
            pl.BlockSpec((r, ND), lambda m: (0, 0)),
            pl.BlockSpec((r, ND), lambda m: (0, 0)),
        ],
        out_shape=[
            jax.ShapeDtypeStruct((r, ND), jnp.float32),
            jax.ShapeDtypeStruct((r, ND), jnp.float32),
        ],
    )(x8, w8a, w8b)


def _worker_chunks():
    wid = lax.axis_index("s") * NC + lax.axis_index("c")
    cnt = CPW + jnp.where(wid < 4, 1, 0)
    cbase = wid * CPW + jnp.minimum(wid, 4)
    cstart = jnp.minimum(cbase, NCHK - (CPW + 1))
    off = cbase - cstart
    return cnt, cbase, off


def _iota16():
    return lax.iota(jnp.int32, 16)


def _gather_body(p_hbm, q_hbm, ei_hbm, out_hbm,
                 idx_v, ra, rb, patch,
                 sa0, sa1, sa2, sa3, sb0, sb1, sb2, sb3,
                 so0, so1, so2, so3):
    sas = (sa0, sa1, sa2, sa3)
    sbs = (sb0, sb1, sb2, sb3)
    sos = (so0, so1, so2, so3)
    cnt, cbase, off = _worker_chunks()
    pltpu.sync_copy(ei_hbm.at[pl.ds(cbase - off, CPW + 1)], idx_v)
    it = _iota16()

    def issue(t, b):
        @pl.when(t < cnt)
        def _():
            pltpu.async_copy(p_hbm.at[idx_v.at[off + t, 0]], ra.at[b], sas[b])
            pltpu.async_copy(q_hbm.at[idx_v.at[off + t, 1]], rb.at[b], sbs[b])

    def drain(d, b):
        @pl.when(d < cnt)
        def _():
            pltpu.make_async_copy(
                p_hbm.at[pl.ds(0, CHE)], ra.at[b], sas[b]).wait()
            pltpu.make_async_copy(
                q_hbm.at[pl.ds(0, CHE)], rb.at[b], sbs[b]).wait()

            @pl.when(d >= NB)
            def _():
                pltpu.make_async_copy(
                    patch.at[b, 0], out_hbm.at[0, 0], sos[b]).wait()
                pltpu.make_async_copy(
                    patch.at[b, 1], out_hbm.at[1, 0], sos[b]).wait()
            rav, rbv = ra.at[b], rb.at[b]
            pav = patch.at[b]

            def per_k(k, c):
                fv = (it + k) & 15
                fd = fv >> 3
                fm = fv & 7
                for grp in range(8):
                    ev = grp * 16 + it
                    va = plsc.load_gather(rav, [ev, fv])
                    vb = plsc.load_gather(rbv, [ev, fv])
                    plsc.store_scatter(pav, [fd, fm, ev], va + vb)
                return c
            lax.fori_loop(0, ED, per_k, 0)
            cb = cbase + d
            pltpu.async_copy(pav.at[0], out_hbm.at[0, cb], sos[b])
            pltpu.async_copy(pav.at[1], out_hbm.at[1, cb], sos[b])

    for b in range(NB - 1):
        issue(b, b)

    def body(g, c):
        for u in range(NB):
            t = g * NB + u
            issue(t + (NB - 1), (u + NB - 1) % NB)
            drain(t, u)
        return c

    lax.fori_loop(0, (CPW + NB) // NB, body, 0)
    for b in range(NB):
        pltpu.make_async_copy(
            patch.at[b, 0], out_hbm.at[0, 0], sos[b]).wait()
        pltpu.make_async_copy(
            patch.at[b, 1], out_hbm.at[1, 0], sos[b]).wait()


def _gather_pq(p, q, ei4):
    f = pl.kernel(
        _gather_body,
        out_type=jax.ShapeDtypeStruct((2, NCHK, 8, ND), jnp.float32),
        mesh=plsc.VectorSubcoreMesh(
            core_axis_name="c", subcore_axis_name="s",
            num_cores=NC, num_subcores=NS),
        scratch_types=[
            pltpu.VMEM((CPW + 1, 2, CHE), jnp.int32),
            pltpu.VMEM((NB, CHE, ED), jnp.float32),
            pltpu.VMEM((NB, CHE, ED), jnp.float32),
            pltpu.VMEM((NB, 2, 8, ND), jnp.float32),
        ] + [pltpu.SemaphoreType.DMA] * (3 * NB),
        compiler_params=pltpu.CompilerParams(use_tc_tiling_on_sc=False, needs_layout_passes=False),
    )
    return f(p, q, ei4)


_CBLK = 50


def _edge_body(g4_ref, eat_ref, we_ref, w2_ref, w3_ref, vb_ref, outt_ref):
    pieces = [g4_ref[:, cb].reshape(ED, ND) for cb in range(_CBLK)]
    g = jnp.concatenate(pieces, axis=1)
    ea = eat_ref[...]
    vb = vb_ref[...]
    b1 = vb[:, 0:1]
    b2 = vb[:, 1:2]
    b3 = vb[:, 2:3]
    gam = vb[:, 3:4]
    bet = vb[:, 4:5]
    t = g + jnp.dot(we_ref[...], ea, preferred_element_type=jnp.float32) + b1
    t = t * jax.nn.sigmoid(t)
    t = jnp.dot(w2_ref[...], t, preferred_element_type=jnp.float32) + b2
    t = t * jax.nn.sigmoid(t)
    t = jnp.dot(w3_ref[...], t, preferred_element_type=jnp.float32) + b3
    mu = jnp.mean(t, axis=0, keepdims=True)
    d = t - mu
    var = jnp.mean(d * d, axis=0, keepdims=True)
    outt_ref[...] = ea + d * lax.rsqrt(var + _EPS) * gam + bet


def _edge_mlp(g4, ea_t, wet, w2t, w3t, vb):
    ne_blk = _CBLK * ND
    full16 = pl.BlockSpec((ED, ED), lambda m: (0, 0))
    return pl.pallas_call(
        _edge_body,
        grid=(NCHK // _CBLK,),
        in_specs=[
            pl.BlockSpec((2, _CBLK, 8, ND), lambda m: (0, m, 0, 0)),
            pl.BlockSpec((ED, ne_blk), lambda m: (0, m)),
            full16, full16, full16,
            pl.BlockSpec((ED, 8), lambda m: (0, 0)),
        ],
        out_specs=pl.BlockSpec((ED, ne_blk), lambda m: (0, m)),
        out_shape=jax.ShapeDtypeStruct((ED, N_EDGES), jnp.float32),
    )(g4, ea_t, wet, w2t, w3t, vb)


def _scatter_body(e4_hbm, ei_hbm, out_hbm,
                  idx_v, ebuf, rbuf, zbuf, acc,
                  se0, se1, se2, se3, ss0, ss1, ss2, ss3):
    ses = (se0, se1, se2, se3)
    sss = (ss0, ss1, ss2, ss3)
    cid = lax.axis_index("c")
    sid = lax.axis_index("s")
    cnt, cbase, off = _worker_chunks()
    it = _iota16()

    def zb(r, c):
        zbuf[r] = jnp.zeros((ED,), jnp.float32)
        return c
    lax.fori_loop(0, 125, zb, 0, unroll=4)
    for k in range(RPT // 125):
        pltpu.sync_copy(zbuf, acc.at[pl.ds(sid * RPT + k * 125, 125)])
    plsc.subcore_barrier()

    pltpu.sync_copy(ei_hbm.at[pl.ds(cbase - off, CPW + 1)], idx_v)

    def issue(t, b):
        @pl.when(t < cnt)
        def _():
            cb = cbase + t
            pltpu.async_copy(e4_hbm.at[0, cb], ebuf.at[b, 0], ses[b])
            pltpu.async_copy(e4_hbm.at[1, cb], ebuf.at[b, 1], ses[b])

    def drain(d, b):
        @pl.when(d < cnt)
        def _():
            pltpu.make_async_copy(
                e4_hbm.at[0, 0], ebuf.at[b, 0], ses[b]).wait()
            pltpu.make_async_copy(
                e4_hbm.at[1, 0], ebuf.at[b, 1], ses[b]).wait()

            @pl.when(d >= NB)
            def _():
                pltpu.make_async_copy(
                    rbuf.at[b], acc.at[idx_v.at[0, 1]], sss[b]).wait()
            ebv, rbv = ebuf.at[b], rbuf.at[b]

            def per_k(k, c):
                fv = (it + k) & 15
                fd = fv >> 3
                fm = fv & 7
                for grp in range(8):
                    ev = grp * 16 + it
                    v = plsc.load_gather(ebv, [fd, fm, ev])
                    plsc.store_scatter(rbv, [ev, fv], v)
                return c
            lax.fori_loop(0, ED, per_k, 0)
            pltpu.async_copy(rbv, acc.at[idx_v.at[off + d, 1]], sss[b],
                             add=True)

    for b in range(NB - 1):
        issue(b, b)

    def body(g, c):
        for u in range(NB):
            t = g * NB + u
            issue(t + (NB - 1), (u + NB - 1) % NB)
            drain(t, u)
        return c

    lax.fori_loop(0, (CPW + NB) // NB, body, 0)
    for b in range(NB):
        pltpu.make_async_copy(
            rbuf.at[b], acc.at[idx_v.at[0, 1]], sss[b]).wait()

    plsc.subcore_barrier()
    pltpu.sync_copy(acc.at[pl.ds(sid * RPT, RPT)],
                    out_hbm.at[cid, pl.ds(sid * RPT, RPT)])


def _scatter_add(e4, ei4):
    f = pl.kernel(
        _scatter_body,
        out_type=jax.ShapeDtypeStruct((NC, N_NODES, ED), jnp.float32),
        mesh=plsc.VectorSubcoreMesh(
            core_axis_name="c", subcore_axis_name="s",
            num_cores=NC, num_subcores=NS),
        scratch_types=[
            pltpu.VMEM((CPW + 1, 2, CHE), jnp.int32),
            pltpu.VMEM((NB, 2, 8, ND), jnp.float32),
            pltpu.VMEM((NB, CHE, ED), jnp.float32),
            pltpu.VMEM((125, ED), jnp.float32),
            pltpu.VMEM_SHARED((N_NODES, ED), jnp.float32),
        ] + [pltpu.SemaphoreType.DMA] * (2 * NB),
        compiler_params=pltpu.CompilerParams(use_tc_tiling_on_sc=False, needs_layout_passes=False),
    )
    return f(e4, ei4)


def _node_body(x_ref, a0_ref, a1_ref, w1a_ref, w1b_ref, b1_ref,
               w2_ref, b2_ref, w3_ref, b3_ref, gam_ref, bet_ref, out_ref):
    x = x_ref[...]
    agg = a0_ref[...] + a1_ref[...]
    t = (jnp.dot(x, w1a_ref[...], preferred_element_type=jnp.float32)
         + jnp.dot(agg, w1b_ref[...], preferred_element_type=jnp.float32)
         + b1_ref[...])
    t = t * jax.nn.sigmoid(t)
    t = jnp.dot(t, w2_ref[...], preferred_element_type=jnp.float32) + b2_ref[...]
    t = t * jax.nn.sigmoid(t)
    t = jnp.dot(t, w3_ref[...], preferred_element_type=jnp.float32) + b3_ref[...]
    mu = jnp.mean(t, axis=-1, keepdims=True)
    d = t - mu
    var = jnp.mean(d * d, axis=-1, keepdims=True)
    out_ref[...] = x + d * lax.rsqrt(var + _EPS) * gam_ref[...] + bet_ref[...]


def _node_mlp(x, a0, a1, w1a, w1b, b1, w2, b2, w3, b3, gam, bet):
    blk = 1000
    full = pl.BlockSpec((ND, ND), lambda m: (0, 0))
    vec = pl.BlockSpec((1, ND), lambda m: (0, 0))
    return pl.pallas_call(
        _node_body,
        grid=(N_NODES // blk,),
        in_specs=[
            pl.BlockSpec((blk, ND), lambda m: (m, 0)),
            pl.BlockSpec((blk, ED), lambda m: (m, 0)),
            pl.BlockSpec((blk, ED), lambda m: (m, 0)),
            full,
            pl.BlockSpec((ED, ND), lambda m: (0, 0)),
            vec, full, vec, full, vec, vec, vec,
        ],
        out_specs=pl.BlockSpec((blk, ND), lambda m: (m, 0)),
        out_shape=jax.ShapeDtypeStruct((N_NODES, ND), jnp.float32),
    )(x, a0, a1, w1a, w1b, b1, w2, b2, w3, b3, gam, bet)


def kernel(x, edge_index, edge_attr,
           eW1, eb1, eW2, eb2, eW3, eb3, e_gamma, e_beta,
           nW1, nb1, nW2, nb2, nW3, nb3, n_gamma, n_beta):
    ei4 = edge_index.astype(jnp.int32).reshape(2, NCHK, CHE).transpose(1, 0, 2)
    eye8 = jnp.eye(8, dtype=jnp.float32)

    x8 = x.reshape(N_NODES // 8, 8 * ND)
    w8a = jnp.kron(eye8, eW1[:ND])
    w8b = jnp.kron(eye8, eW1[ND:2 * ND])
    p_pk, q_pk = _project_pq(x8, w8a, w8b)

    g4 = _gather_pq(p_pk.reshape(N_NODES, ED), q_pk.reshape(N_NODES, ED), ei4)

    vb = jnp.stack([eb1, eb2, eb3, e_gamma, e_beta], axis=1)
    vb = jnp.concatenate([vb, jnp.zeros((ED, 3), jnp.float32)], axis=1)
    e_new_t = _edge_mlp(g4, edge_attr.T, eW1[2 * ND:].T, eW2.T, eW3.T, vb)

    e4 = e_new_t.reshape(2, 8, NCHK, CHE).transpose(0, 2, 1, 3)
    aggp = _scatter_add(e4, ei4)

    x_new = _node_mlp(
        x, aggp[0], aggp[1],
        nW1[:ND], nW1[ND:], nb1.reshape(1, ND),
        nW2, nb2.reshape(1, ND), nW3, nb3.reshape(1, ND),
        n_gamma.reshape(1, ND), n_beta.reshape(1, ND))

    return (x_new, e_new_t.T)
